# Initial kernel scaffold; baseline (speedup 1.0000x reference)
import jax
import jax.numpy as jnp
from jax.experimental import pallas as pl


def kernel(x, w0, w1, w2, g0, g1, g2, g3):
    raise NotImplementedError("write your pallas kernel here")


# speedup vs baseline: 1.6046x; 1.6046x over previous
import functools

import jax
import jax.numpy as jnp
from jax.experimental import pallas as pl
from jax.experimental.pallas import tpu as pltpu

EPS = 1e-6
H = 4096
M_TILE = 256
N_SLICE = 1024


def _layer_kernel(x_ref, w_ref, gin_ref, gout_ref, o_ref, *, relu_in, final_norm):
    xb = x_ref[...]
    if relu_in:
        xb = jnp.maximum(xb, 0.0)
    var = jnp.mean(xb * xb, axis=1, keepdims=True)
    y = (xb * jax.lax.rsqrt(var + EPS) * gin_ref[...]).astype(jnp.bfloat16)
    for n in range(0, H, N_SLICE):
        sl = slice(n, n + N_SLICE)
        acc = jnp.dot(y, w_ref[:, sl], preferred_element_type=jnp.float32)
        o_ref[:, sl] = acc + xb[:, sl]
    if final_norm:
        r = o_ref[...]
        var2 = jnp.mean(r * r, axis=1, keepdims=True)
        o_ref[...] = r * jax.lax.rsqrt(var2 + EPS) * gout_ref[...]


def _layer(x, w_bf16, g_in, g_out, *, relu_in, final_norm, interpret=False):
    t = x.shape[0]
    body = functools.partial(_layer_kernel, relu_in=relu_in, final_norm=final_norm)
    return pl.pallas_call(
        body,
        out_shape=jax.ShapeDtypeStruct((t, H), jnp.float32),
        grid=(t // M_TILE,),
        in_specs=[
            pl.BlockSpec((M_TILE, H), lambda i: (i, 0)),
            pl.BlockSpec((H, H), lambda i: (0, 0)),
            pl.BlockSpec((1, H), lambda i: (0, 0)),
            pl.BlockSpec((1, H), lambda i: (0, 0)),
        ],
        out_specs=pl.BlockSpec((M_TILE, H), lambda i: (i, 0)),
        compiler_params=pltpu.CompilerParams(
            dimension_semantics=("parallel",),
            vmem_limit_bytes=56 * 1024 * 1024,
        ),
        name=f"rmsnorm_mm_{'relu' if relu_in else 'mid' if not final_norm else 'fin'}",
        interpret=interpret,
    )(x, w_bf16, g_in, g_out)


def kernel(x, w0, w1, w2, g0, g1, g2, g3, *, interpret=False):
    w0b = w0.astype(jnp.bfloat16)
    w1b = w1.astype(jnp.bfloat16)
    w2b = w2.astype(jnp.bfloat16)
    g0r = g0.reshape(1, H)
    g1r = g1.reshape(1, H)
    g2r = g2.reshape(1, H)
    g3r = g3.reshape(1, H)
    r1 = _layer(x, w0b, g0r, g1r, relu_in=True, final_norm=False, interpret=interpret)
    r2 = _layer(r1, w1b, g1r, g2r, relu_in=False, final_norm=False, interpret=interpret)
    y4 = _layer(r2, w2b, g2r, g3r, relu_in=False, final_norm=True, interpret=interpret)
    return y4
